# baseline (speedup 1.0000x reference)
import dataclasses
import functools

import jax
import jax.numpy as jnp
from jax import lax
from jax.experimental import pallas as pl
from jax.experimental.pallas import tpu as pltpu
from jax.experimental.pallas import tpu_sc as plsc

VOCAB = 100000
EMBED = 128
B = 4
S = 2048
N = B * S
MAX_POS = 4096
LN_EPS = 1e-5

NUM_CORES = 2
NUM_SUBCORES = 16
NW = NUM_CORES * NUM_SUBCORES
ROWS_PER_W = N // NW
S_PER_W = S // NW
L = 16


def _sc_gather(token_table, input_ids):
    mesh = plsc.VectorSubcoreMesh(core_axis_name="c", subcore_axis_name="s")
    cp = pltpu.CompilerParams()
    if "needs_layout_passes" in pltpu.CompilerParams.__dataclass_fields__:
        cp = dataclasses.replace(cp, needs_layout_passes=False)

    @functools.partial(
        pl.kernel,
        mesh=mesh,
        compiler_params=cp,
        out_type=jax.ShapeDtypeStruct((N, EMBED), jnp.float32),
        scratch_types=[
            pltpu.VMEM((B, S_PER_W), jnp.int32),
            pltpu.VMEM((ROWS_PER_W,), jnp.int32),
            pltpu.VMEM((ROWS_PER_W, EMBED), jnp.float32),
            pltpu.SemaphoreType.DMA,
        ],
    )
    def k(ids_hbm, table_hbm, out_hbm, idsb_v, idx_v, rows_v, sem):
        wid = lax.axis_index("s") * NUM_CORES + lax.axis_index("c")
        base = wid * ROWS_PER_W
        s0 = wid * S_PER_W
        for b in range(B):
            pltpu.sync_copy(ids_hbm.at[b, pl.ds(s0, S_PER_W)], idsb_v.at[b])
        lanes = lax.iota(jnp.int32, L)
        for j in range(ROWS_PER_W // L):
            r = j * L + lanes
            idx_v[pl.ds(j * L, L)] = plsc.load_gather(
                idsb_v, [r & (B - 1), lax.shift_right_logical(r, 2)]
            )
        pltpu.async_copy(table_hbm.at[idx_v], rows_v, sem).wait()
        pltpu.sync_copy(rows_v, out_hbm.at[pl.ds(base, ROWS_PER_W)])

    return k(input_ids, token_table)


S_BLK = 1024


def _tc_ln_body(x_ref, pos_ref, g_ref, b_ref, o_ref):
    x = x_ref[...].reshape(S_BLK, B, EMBED)
    i = pl.program_id(0)
    p = pos_ref[pl.ds(2 + i * S_BLK, S_BLK), :]
    e = x + p[:, None, :]
    mean = jnp.mean(e, axis=-1, keepdims=True)
    c = e - mean
    var = jnp.mean(c * c, axis=-1, keepdims=True)
    o_ref[...] = c * lax.rsqrt(var + LN_EPS) * g_ref[...] + b_ref[...]


def _tc_ln(gathered2d, position_table, ln_gamma, ln_beta):
    return pl.pallas_call(
        _tc_ln_body,
        grid=(S // S_BLK,),
        in_specs=[
            pl.BlockSpec((S_BLK * B, EMBED), lambda i: (i, 0)),
            pl.BlockSpec((MAX_POS + 2, EMBED), lambda i: (0, 0)),
            pl.BlockSpec((EMBED,), lambda i: (0,)),
            pl.BlockSpec((EMBED,), lambda i: (0,)),
        ],
        out_specs=pl.BlockSpec((S_BLK, B, EMBED), lambda i: (i, 0, 0)),
        out_shape=jax.ShapeDtypeStruct((S, B, EMBED), jnp.float32),
    )(gathered2d, position_table, ln_gamma, ln_beta)


def kernel(input_ids, token_table, position_table, ln_gamma, ln_beta):
    gathered = _sc_gather(token_table, input_ids.astype(jnp.int32))
    return _tc_ln(gathered, position_table, ln_gamma, ln_beta)


# speedup vs baseline: 1.0058x; 1.0058x over previous
import functools

import jax
import jax.numpy as jnp
from jax import lax
from jax.experimental import pallas as pl
from jax.experimental.pallas import tpu as pltpu
from jax.experimental.pallas import tpu_sc as plsc

VOCAB = 100000
EMBED = 128
B = 4
S = 2048
N = B * S
LN_EPS = 1e-5

NUM_CORES = 2
NUM_SUBCORES = 16
NW = NUM_CORES * NUM_SUBCORES
ROWS_PER_W = N // NW


def _sc_gather(token_table, ids_flat):
    mesh = plsc.VectorSubcoreMesh(core_axis_name="c", subcore_axis_name="s")

    @functools.partial(
        pl.kernel,
        mesh=mesh,
        out_type=jax.ShapeDtypeStruct((N, EMBED), jnp.float32),
        scratch_types=[
            pltpu.VMEM((ROWS_PER_W,), jnp.int32),
            pltpu.VMEM((ROWS_PER_W, EMBED), jnp.float32),
            pltpu.SemaphoreType.DMA,
        ],
    )
    def k(ids_hbm, table_hbm, out_hbm, idx_v, rows_v, sem):
        wid = lax.axis_index("s") * NUM_CORES + lax.axis_index("c")
        base = wid * ROWS_PER_W
        pltpu.sync_copy(ids_hbm.at[pl.ds(base, ROWS_PER_W)], idx_v)
        pltpu.async_copy(table_hbm.at[idx_v], rows_v, sem).wait()
        pltpu.sync_copy(rows_v, out_hbm.at[pl.ds(base, ROWS_PER_W)])

    return k(ids_flat, token_table)


S_BLK = 256


def _tc_ln_body(x_ref, pos_ref, g_ref, b_ref, o_ref):
    x = x_ref[...].reshape(S_BLK, B, EMBED)
    p = pos_ref[...]
    e = x + p[:, None, :]
    mean = jnp.mean(e, axis=-1, keepdims=True)
    c = e - mean
    var = jnp.mean(c * c, axis=-1, keepdims=True)
    o_ref[...] = c * lax.rsqrt(var + LN_EPS) * g_ref[...] + b_ref[...]


def _tc_ln(gathered2d, pos, ln_gamma, ln_beta):
    return pl.pallas_call(
        _tc_ln_body,
        grid=(S // S_BLK,),
        in_specs=[
            pl.BlockSpec((S_BLK * B, EMBED), lambda i: (i, 0)),
            pl.BlockSpec((S_BLK, EMBED), lambda i: (i, 0)),
            pl.BlockSpec((EMBED,), lambda i: (0,)),
            pl.BlockSpec((EMBED,), lambda i: (0,)),
        ],
        out_specs=pl.BlockSpec((S_BLK, B, EMBED), lambda i: (i, 0, 0)),
        out_shape=jax.ShapeDtypeStruct((S, B, EMBED), jnp.float32),
    )(gathered2d, pos, ln_gamma, ln_beta)


def kernel(input_ids, token_table, position_table, ln_gamma, ln_beta):
    ids_flat = input_ids.astype(jnp.int32).T.reshape(-1)
    gathered = _sc_gather(token_table, ids_flat)
    pos = lax.slice(position_table, (2, 0), (2 + S, EMBED))
    return _tc_ln(gathered, pos, ln_gamma, ln_beta)


# speedup vs baseline: 1.0546x; 1.0485x over previous
import functools

import jax
import jax.numpy as jnp
from jax import lax
from jax.experimental import pallas as pl
from jax.experimental.pallas import tpu as pltpu
from jax.experimental.pallas import tpu_sc as plsc

VOCAB = 100000
EMBED = 128
B = 4
S = 2048
N = B * S
LN_EPS = 1e-5

NUM_CORES = 2
NUM_SUBCORES = 16
NW = NUM_CORES * NUM_SUBCORES
ROWS_PER_W = N // NW


def _sc_gather(token_table, ids_flat):
    mesh = plsc.VectorSubcoreMesh(core_axis_name="c", subcore_axis_name="s")

    @functools.partial(
        pl.kernel,
        mesh=mesh,
        out_type=jax.ShapeDtypeStruct((N, EMBED), jnp.float32),
        scratch_types=[
            pltpu.VMEM((ROWS_PER_W,), jnp.int32),
            pltpu.VMEM((ROWS_PER_W, EMBED), jnp.float32),
            pltpu.SemaphoreType.DMA,
        ],
    )
    def k(ids_hbm, table_hbm, out_hbm, idx_v, rows_v, sem):
        wid = lax.axis_index("s") * NUM_CORES + lax.axis_index("c")
        base = wid * ROWS_PER_W
        pltpu.sync_copy(ids_hbm.at[pl.ds(base, ROWS_PER_W)], idx_v)
        pltpu.async_copy(table_hbm.at[idx_v], rows_v, sem).wait()
        pltpu.sync_copy(rows_v, out_hbm.at[pl.ds(base, ROWS_PER_W)])

    return k(ids_flat, token_table)


S_BLK = 512


def _tc_ln_body(x_ref, pos_ref, g_ref, b_ref, o_ref):
    x = x_ref[...].reshape(S_BLK, B, EMBED)
    p = pos_ref[...]
    e = x + p[:, None, :]
    mean = jnp.mean(e, axis=-1, keepdims=True)
    c = e - mean
    var = jnp.mean(c * c, axis=-1, keepdims=True)
    o_ref[...] = c * lax.rsqrt(var + LN_EPS) * g_ref[...] + b_ref[...]


def _tc_ln(gathered2d, pos, ln_gamma, ln_beta):
    return pl.pallas_call(
        _tc_ln_body,
        grid=(S // S_BLK,),
        in_specs=[
            pl.BlockSpec((S_BLK * B, EMBED), lambda i: (i, 0)),
            pl.BlockSpec((S_BLK, EMBED), lambda i: (i, 0)),
            pl.BlockSpec((EMBED,), lambda i: (0,)),
            pl.BlockSpec((EMBED,), lambda i: (0,)),
        ],
        out_specs=pl.BlockSpec((S_BLK, B, EMBED), lambda i: (i, 0, 0)),
        out_shape=jax.ShapeDtypeStruct((S, B, EMBED), jnp.float32),
    )(gathered2d, pos, ln_gamma, ln_beta)


def kernel(input_ids, token_table, position_table, ln_gamma, ln_beta):
    ids_flat = input_ids.astype(jnp.int32).T.reshape(-1)
    gathered = _sc_gather(token_table, ids_flat)
    pos = lax.slice(position_table, (2, 0), (2 + S, EMBED))
    return _tc_ln(gathered, pos, ln_gamma, ln_beta)
